# baseline (speedup 1.0000x reference)
import functools

import jax
import jax.numpy as jnp
from jax import lax
from jax.experimental import pallas as pl
from jax.experimental.pallas import tpu as pltpu
from jax.experimental.pallas import tpu_sc as plsc

BATCH = 16384
DIM = 64
L = 16
NC = 2
NS = 16
NW = NC * NS
BPW = BATCH // NW
CHUNK = 128
NCHUNK = BPW // CHUNK

_EPS2 = 1e-24


def _rsqrt_nr(x):
    i = plsc.bitcast(x, jnp.int32)
    i = jnp.int32(0x5F3759DF) - (i >> 1)
    y = plsc.bitcast(i, jnp.float32)
    half_x = 0.5 * x
    for _ in range(3):
        y = y * (1.5 - half_x * y * y)
    return y


def _body(uids, iids, utab, itab, out,
          idx_u, idx_i, rows_u, rows_i, scores, sem):
    wid = lax.axis_index("s") * NC + lax.axis_index("c")
    base = wid * NCHUNK

    pltpu.sync_copy(uids.at[pl.ds(base, NCHUNK)], idx_u)
    pltpu.sync_copy(iids.at[pl.ds(base, NCHUNK)], idx_i)

    copies = []
    for j in range(NCHUNK):
        copies.append(pltpu.async_copy(
            utab.at[idx_u.at[j]], rows_u.at[pl.ds(j * CHUNK, CHUNK)], sem))
        copies.append(pltpu.async_copy(
            itab.at[idx_i.at[j]], rows_i.at[pl.ds(j * CHUNK, CHUNK)], sem))
    for c in copies:
        c.wait()

    lanes = lax.broadcasted_iota(jnp.int32, (L,), 0)

    def grp_body(g, _):
        off = g * L
        dotv = jnp.zeros((L,), jnp.float32)
        nuv = jnp.zeros((L,), jnp.float32)
        niv = jnp.zeros((L,), jnp.float32)
        for r in range(L):
            row = off + r
            p = jnp.zeros((L,), jnp.float32)
            qu = jnp.zeros((L,), jnp.float32)
            qi = jnp.zeros((L,), jnp.float32)
            for d in range(DIM // L):
                u = rows_u[row, pl.ds(d * L, L)]
                v = rows_i[row, pl.ds(d * L, L)]
                p = p + u * v
                qu = qu + u * u
                qi = qi + v * v
            m = lanes == r
            dotv = jnp.where(m, jnp.sum(p), dotv)
            nuv = jnp.where(m, jnp.sum(qu), nuv)
            niv = jnp.where(m, jnp.sum(qi), niv)
        nuv = jnp.maximum(nuv, _EPS2)
        niv = jnp.maximum(niv, _EPS2)
        scores[pl.ds(off, L)] = dotv * _rsqrt_nr(nuv) * _rsqrt_nr(niv)
        return ()

    lax.fori_loop(0, BPW // L, grp_body, ())

    pltpu.sync_copy(scores, out.at[pl.ds(wid * BPW, BPW)])


@jax.jit
def _run(uids, iids, utab, itab):
    mesh = plsc.VectorSubcoreMesh(core_axis_name="c", subcore_axis_name="s")
    return pl.kernel(
        _body,
        mesh=mesh,
        compiler_params=pltpu.CompilerParams(
            needs_layout_passes=False, use_tc_tiling_on_sc=False),
        out_type=jax.ShapeDtypeStruct((BATCH,), jnp.float32),
        scratch_types=[
            pltpu.VMEM((NCHUNK, CHUNK), jnp.int32),
            pltpu.VMEM((NCHUNK, CHUNK), jnp.int32),
            pltpu.VMEM((BPW, DIM), jnp.float32),
            pltpu.VMEM((BPW, DIM), jnp.float32),
            pltpu.VMEM((BPW,), jnp.float32),
            pltpu.SemaphoreType.DMA,
        ],
    )(uids, iids, utab, itab)


def kernel(user_ids, item_ids, user_table, item_table):
    uids = user_ids.astype(jnp.int32).reshape(NW * NCHUNK, CHUNK)
    iids = item_ids.astype(jnp.int32).reshape(NW * NCHUNK, CHUNK)
    return _run(uids, iids, user_table, item_table)
